# Initial kernel scaffold; baseline (speedup 1.0000x reference)
import jax
import jax.numpy as jnp
from jax.experimental import pallas as pl


def kernel(node_features, node_attr, edge_src, edge_dst, edge_attr, edge_scalars, W_sc, W_lin1, W_lin2, W_alpha, fc_w0, fc_w1):
    raise NotImplementedError("write your pallas kernel here")


# speedup vs baseline: 1.3171x; 1.3171x over previous
import jax
import jax.numpy as jnp
import numpy as np
from jax import lax
from jax.experimental import pallas as pl
from jax.experimental.pallas import tpu as pltpu
from jax.experimental.pallas import tpu_sc as plsc

_z = np.random.RandomState(0).randn(1_000_000)
_silu_np = _z / (1.0 + np.exp(-_z))
_SILU_CST = float(1.0 / np.sqrt(np.mean(_silu_np ** 2)))

_N = 10000
_E = 320000
_D = 128
_DH = 64
_DS = 16
_FH = 64
_INV_SQRT_DS = 1.0 / np.sqrt(float(_DS))
_INV_SQRT_FH = 1.0 / np.sqrt(float(_FH))
_INV_SQRT_D = 1.0 / np.sqrt(float(_D))
_INV_SQRT_NB = 1.0 / np.sqrt(32.0)

_NC = 2
_NS = 16
_E_PER_SUB = _E // _NS
_CHUNK = 80
_NCHUNKS = _E_PER_SUB // _CHUNK
_N_PAD = 10240
_ROWS_PER_SUB = _N_PAD // _NS


def _edge_mlp_body(es_ref, ea_ref, w0_ref, w1_ref, out_ref):
    h = jnp.dot(es_ref[...], w0_ref[...] * _INV_SQRT_DS,
                preferred_element_type=jnp.float32)
    h = jax.nn.silu(h) * _SILU_CST
    w = jnp.dot(h, w1_ref[0] * _INV_SQRT_FH,
                preferred_element_type=jnp.float32)
    out_ref[...] = w * ea_ref[...] * _INV_SQRT_NB


def _edge_mlp(edge_scalars, edge_attr, fc_w0, fc_w1_halves):
    be = 2000
    gi = _E // be
    return pl.pallas_call(
        _edge_mlp_body,
        grid=(_NC, gi),
        in_specs=[
            pl.BlockSpec((be, _DS), lambda c, i: (i, 0)),
            pl.BlockSpec((be, 1), lambda c, i: (i, 0)),
            pl.BlockSpec((_DS, _FH), lambda c, i: (0, 0)),
            pl.BlockSpec((1, _FH, _DH), lambda c, i: (c, 0, 0)),
        ],
        out_specs=pl.BlockSpec((be, _DH), lambda c, i: (c * gi + i, 0)),
        out_shape=jax.ShapeDtypeStruct((_NC * _E, _DH), jnp.float32),
    )(edge_scalars, edge_attr, fc_w0, fc_w1_halves)


def _node_lin_body(nf_ref, na_ref, w_ref, out_ref):
    xf = jnp.dot(nf_ref[...], w_ref[0], preferred_element_type=jnp.float32)
    out_ref[...] = xf * na_ref[...] * _INV_SQRT_D


def _node_lin(node_features, node_attr, W_lin1_halves):
    bn = 2000
    gi = _N // bn
    return pl.pallas_call(
        _node_lin_body,
        grid=(_NC, gi),
        in_specs=[
            pl.BlockSpec((bn, _D), lambda c, i: (i, 0)),
            pl.BlockSpec((bn, 1), lambda c, i: (i, 0)),
            pl.BlockSpec((1, _D, _DH), lambda c, i: (c, 0, 0)),
        ],
        out_specs=pl.BlockSpec((bn, _DH), lambda c, i: (c * gi + i, 0)),
        out_shape=jax.ShapeDtypeStruct((_NC * _N, _DH), jnp.float32),
    )(node_features, node_attr, W_lin1_halves)


def _final_body(nf_ref, na_ref, p_ref, wsc_ref, wl2_ref, wal_ref, out_ref):
    na = na_ref[...]
    p0 = p_ref[0]
    p1 = p_ref[1]
    nsc = jnp.dot(nf_ref[...], wsc_ref[...],
                  preferred_element_type=jnp.float32) * na * _INV_SQRT_D
    wl2 = wl2_ref[...]
    nco = (jnp.dot(p0, wl2[:_DH, :], preferred_element_type=jnp.float32)
           + jnp.dot(p1, wl2[_DH:, :], preferred_element_type=jnp.float32))
    nco = nco * na * _INV_SQRT_D
    wal = wal_ref[...]
    alpha = (jnp.dot(p0, wal[:_DH, :], preferred_element_type=jnp.float32)
             + jnp.dot(p1, wal[_DH:, :], preferred_element_type=jnp.float32))
    alpha = alpha * na * _INV_SQRT_D
    out_ref[...] = jax.nn.silu(nsc + alpha * nco) * _SILU_CST


def _final(node_features, node_attr, parts, W_sc, W_lin2, W_alpha):
    bn = 2000
    grid = _N // bn
    return pl.pallas_call(
        _final_body,
        grid=(grid,),
        in_specs=[
            pl.BlockSpec((bn, _D), lambda i: (i, 0)),
            pl.BlockSpec((bn, 1), lambda i: (i, 0)),
            pl.BlockSpec((_NC, bn, _DH), lambda i: (0, i, 0)),
            pl.BlockSpec((_D, _D), lambda i: (0, 0)),
            pl.BlockSpec((_D, _D), lambda i: (0, 0)),
            pl.BlockSpec((_D, 1), lambda i: (0, 0)),
        ],
        out_specs=pl.BlockSpec((bn, _D), lambda i: (i, 0)),
        out_shape=jax.ShapeDtypeStruct((_N, _D), jnp.float32),
    )(node_features, node_attr, parts, W_sc, W_lin2, W_alpha)


def _sc_body(xf_hbm, w_hbm, src_hbm, dst_hbm, out_hbm,
             src_v, dst_v, rows_v, wchunk_v, agg_sh):
    cid = lax.axis_index("c")
    sid = lax.axis_index("s")

    pltpu.sync_copy(src_hbm.at[sid], src_v)
    pltpu.sync_copy(dst_hbm.at[sid], dst_v)

    off = cid * _N

    def off_body(c, carry):
        for j in range(_CHUNK // 16):
            sl = pl.ds(j * 16, 16)
            src_v[c, sl] = src_v[c, sl] + off
        return carry

    lax.fori_loop(0, _NCHUNKS, off_body, 0)

    def z_body(e, carry):
        for j in range(_DH // 16):
            rows_v[e, pl.ds(j * 16, 16)] = jnp.zeros((16,), jnp.float32)
        return carry

    lax.fori_loop(0, _CHUNK, z_body, 0)
    row0 = sid * _ROWS_PER_SUB
    for k in range(_ROWS_PER_SUB // _CHUNK):
        pltpu.sync_copy(rows_v, agg_sh.at[pl.ds(row0 + k * _CHUNK, _CHUNK)])
    plsc.subcore_barrier()

    def chunk_body(c, carry):
        pltpu.sync_copy(xf_hbm.at[src_v.at[c]], rows_v)
        pltpu.sync_copy(w_hbm.at[cid * _NS + sid, c], wchunk_v)

        def mul_body(e, carry2):
            for j in range(_DH // 16):
                sl = pl.ds(j * 16, 16)
                rows_v[e, sl] = rows_v[e, sl] * wchunk_v[e, sl]
            return carry2

        lax.fori_loop(0, _CHUNK, mul_body, 0)

        pltpu.sync_copy(rows_v, agg_sh.at[dst_v.at[c]], add=True)
        return carry

    lax.fori_loop(0, _NCHUNKS, chunk_body, 0)
    plsc.subcore_barrier()

    pltpu.sync_copy(agg_sh.at[pl.ds(row0, _ROWS_PER_SUB)],
                    out_hbm.at[cid, pl.ds(row0, _ROWS_PER_SUB)])


def _sc_gather_scatter(xf_flat, w_halves, src3, dst3):
    mesh = plsc.VectorSubcoreMesh(core_axis_name="c", subcore_axis_name="s")
    f = pl.kernel(
        _sc_body,
        out_type=jax.ShapeDtypeStruct((_NC, _N_PAD, _DH), jnp.float32),
        mesh=mesh,
        compiler_params=pltpu.CompilerParams(use_tc_tiling_on_sc=False),
        scratch_types=[
            pltpu.VMEM((_NCHUNKS, _CHUNK), jnp.int32),
            pltpu.VMEM((_NCHUNKS, _CHUNK), jnp.int32),
            pltpu.VMEM((_CHUNK, _DH), jnp.float32),
            pltpu.VMEM((_CHUNK, _DH), jnp.float32),
            pltpu.VMEM_SHARED((_N_PAD, _DH), jnp.float32),
        ],
    )
    return f(xf_flat, w_halves, src3, dst3)


def kernel(node_features, node_attr, edge_src, edge_dst, edge_attr,
           edge_scalars, W_sc, W_lin1, W_lin2, W_alpha, fc_w0, fc_w1):
    fc_w1_h = jnp.stack([fc_w1[:, :_DH], fc_w1[:, _DH:]], axis=0)
    W_lin1_h = jnp.stack([W_lin1[:, :_DH], W_lin1[:, _DH:]], axis=0)
    w_halves = _edge_mlp(edge_scalars, edge_attr, fc_w0, fc_w1_h)
    xf_flat = _node_lin(node_features, node_attr, W_lin1_h)

    src3 = edge_src.reshape(_NS, _NCHUNKS, _CHUNK)
    dst3 = edge_dst.reshape(_NS, _NCHUNKS, _CHUNK)
    w4 = w_halves.reshape(_NC * _NS, _NCHUNKS, _CHUNK, _DH)

    parts = _sc_gather_scatter(xf_flat, w4, src3, dst3)
    return _final(node_features, node_attr, parts, W_sc, W_lin2, W_alpha)


# speedup vs baseline: 1.4942x; 1.1345x over previous
import jax
import jax.numpy as jnp
import numpy as np
from jax import lax
from jax.experimental import pallas as pl
from jax.experimental.pallas import tpu as pltpu
from jax.experimental.pallas import tpu_sc as plsc

_z = np.random.RandomState(0).randn(1_000_000)
_silu_np = _z / (1.0 + np.exp(-_z))
_SILU_CST = float(1.0 / np.sqrt(np.mean(_silu_np ** 2)))

_N = 10000
_E = 320000
_D = 128
_DH = 64
_DS = 16
_FH = 64
_INV_SQRT_DS = 1.0 / np.sqrt(float(_DS))
_INV_SQRT_FH = 1.0 / np.sqrt(float(_FH))
_INV_SQRT_D = 1.0 / np.sqrt(float(_D))
_INV_SQRT_NB = 1.0 / np.sqrt(32.0)

_NC = 2
_NS = 16
_E_PER_SUB = _E // _NS
_CHUNK = 80
_NCHUNKS = _E_PER_SUB // _CHUNK
_N_PAD = 10240
_ROWS_PER_SUB = _N_PAD // _NS


_BE = 2000
_BCH = _BE // _CHUNK
_GJ = _E_PER_SUB // _BE


def _edge_mlp_body(es_ref, ea_ref, w0_ref, w1_ref, out_ref):
    h = jnp.dot(es_ref[...], w0_ref[...] * _INV_SQRT_DS,
                preferred_element_type=jnp.float32)
    h = jax.nn.silu(h) * _SILU_CST
    ea = ea_ref[...] * _INV_SQRT_NB
    w0 = jnp.dot(h, w1_ref[0] * _INV_SQRT_FH,
                 preferred_element_type=jnp.float32) * ea
    w1 = jnp.dot(h, w1_ref[1] * _INV_SQRT_FH,
                 preferred_element_type=jnp.float32) * ea
    out_ref[0, 0] = w0.reshape(_BCH, _CHUNK, _DH)
    out_ref[1, 0] = w1.reshape(_BCH, _CHUNK, _DH)


def _edge_mlp(edge_scalars, edge_attr, fc_w0, fc_w1_halves):
    return pl.pallas_call(
        _edge_mlp_body,
        grid=(_NS, _GJ),
        in_specs=[
            pl.BlockSpec((_BE, _DS), lambda s, j: (s * _GJ + j, 0)),
            pl.BlockSpec((_BE, 1), lambda s, j: (s * _GJ + j, 0)),
            pl.BlockSpec((_DS, _FH), lambda s, j: (0, 0)),
            pl.BlockSpec((_NC, _FH, _DH), lambda s, j: (0, 0, 0)),
        ],
        out_specs=pl.BlockSpec((_NC, 1, _BCH, _CHUNK, _DH),
                               lambda s, j: (0, s, j, 0, 0)),
        out_shape=jax.ShapeDtypeStruct((_NC, _NS, _NCHUNKS, _CHUNK, _DH),
                                       jnp.float32),
    )(edge_scalars, edge_attr, fc_w0, fc_w1_halves)


def _node_lin_body(nf_ref, na_ref, w_ref, out_ref):
    xf = jnp.dot(nf_ref[...], w_ref[0], preferred_element_type=jnp.float32)
    out_ref[...] = xf * na_ref[...] * _INV_SQRT_D


def _node_lin(node_features, node_attr, W_lin1_halves):
    bn = 2000
    gi = _N // bn
    return pl.pallas_call(
        _node_lin_body,
        grid=(_NC, gi),
        in_specs=[
            pl.BlockSpec((bn, _D), lambda c, i: (i, 0)),
            pl.BlockSpec((bn, 1), lambda c, i: (i, 0)),
            pl.BlockSpec((1, _D, _DH), lambda c, i: (c, 0, 0)),
        ],
        out_specs=pl.BlockSpec((bn, _DH), lambda c, i: (c * gi + i, 0)),
        out_shape=jax.ShapeDtypeStruct((_NC * _N, _DH), jnp.float32),
    )(node_features, node_attr, W_lin1_halves)


def _final_body(nf_ref, na_ref, p_ref, wsc_ref, wl2_ref, wal_ref, out_ref):
    na = na_ref[...]
    p0 = p_ref[0]
    p1 = p_ref[1]
    nsc = jnp.dot(nf_ref[...], wsc_ref[...],
                  preferred_element_type=jnp.float32) * na * _INV_SQRT_D
    wl2 = wl2_ref[...]
    nco = (jnp.dot(p0, wl2[:_DH, :], preferred_element_type=jnp.float32)
           + jnp.dot(p1, wl2[_DH:, :], preferred_element_type=jnp.float32))
    nco = nco * na * _INV_SQRT_D
    wal = wal_ref[...]
    alpha = (jnp.dot(p0, wal[:_DH, :], preferred_element_type=jnp.float32)
             + jnp.dot(p1, wal[_DH:, :], preferred_element_type=jnp.float32))
    alpha = alpha * na * _INV_SQRT_D
    out_ref[...] = jax.nn.silu(nsc + alpha * nco) * _SILU_CST


def _final(node_features, node_attr, parts, W_sc, W_lin2, W_alpha):
    bn = 2000
    grid = _N // bn
    return pl.pallas_call(
        _final_body,
        grid=(grid,),
        in_specs=[
            pl.BlockSpec((bn, _D), lambda i: (i, 0)),
            pl.BlockSpec((bn, 1), lambda i: (i, 0)),
            pl.BlockSpec((_NC, bn, _DH), lambda i: (0, i, 0)),
            pl.BlockSpec((_D, _D), lambda i: (0, 0)),
            pl.BlockSpec((_D, _D), lambda i: (0, 0)),
            pl.BlockSpec((_D, 1), lambda i: (0, 0)),
        ],
        out_specs=pl.BlockSpec((bn, _D), lambda i: (i, 0)),
        out_shape=jax.ShapeDtypeStruct((_N, _D), jnp.float32),
    )(node_features, node_attr, parts, W_sc, W_lin2, W_alpha)


def _sc_body(xf_hbm, w_hbm, src_hbm, dst_hbm, out_hbm,
             src_v, dst_v, rows_v, wchunk_v, agg_sh):
    cid = lax.axis_index("c")
    sid = lax.axis_index("s")

    pltpu.sync_copy(src_hbm.at[sid], src_v)
    pltpu.sync_copy(dst_hbm.at[sid], dst_v)

    off = cid * _N

    def off_body(c, carry):
        for j in range(_CHUNK // 16):
            sl = pl.ds(j * 16, 16)
            src_v[c, sl] = src_v[c, sl] + off
        return carry

    lax.fori_loop(0, _NCHUNKS, off_body, 0)

    def z_body(e, carry):
        for j in range(_DH // 16):
            rows_v[e, pl.ds(j * 16, 16)] = jnp.zeros((16,), jnp.float32)
        return carry

    lax.fori_loop(0, _CHUNK, z_body, 0)
    row0 = sid * _ROWS_PER_SUB
    for k in range(_ROWS_PER_SUB // _CHUNK):
        pltpu.sync_copy(rows_v, agg_sh.at[pl.ds(row0 + k * _CHUNK, _CHUNK)])
    plsc.subcore_barrier()

    def chunk_body(c, carry):
        pltpu.sync_copy(xf_hbm.at[src_v.at[c]], rows_v)
        pltpu.sync_copy(w_hbm.at[cid, sid, c], wchunk_v)

        def mul_body(e, carry2):
            for j in range(_DH // 16):
                sl = pl.ds(j * 16, 16)
                rows_v[e, sl] = rows_v[e, sl] * wchunk_v[e, sl]
            return carry2

        lax.fori_loop(0, _CHUNK, mul_body, 0)

        pltpu.sync_copy(rows_v, agg_sh.at[dst_v.at[c]], add=True)
        return carry

    lax.fori_loop(0, _NCHUNKS, chunk_body, 0)
    plsc.subcore_barrier()

    pltpu.sync_copy(agg_sh.at[pl.ds(row0, _ROWS_PER_SUB)],
                    out_hbm.at[cid, pl.ds(row0, _ROWS_PER_SUB)])


def _sc_gather_scatter(xf_flat, w_halves, src3, dst3):
    mesh = plsc.VectorSubcoreMesh(core_axis_name="c", subcore_axis_name="s")
    f = pl.kernel(
        _sc_body,
        out_type=jax.ShapeDtypeStruct((_NC, _N_PAD, _DH), jnp.float32),
        mesh=mesh,
        compiler_params=pltpu.CompilerParams(use_tc_tiling_on_sc=False),
        scratch_types=[
            pltpu.VMEM((_NCHUNKS, _CHUNK), jnp.int32),
            pltpu.VMEM((_NCHUNKS, _CHUNK), jnp.int32),
            pltpu.VMEM((_CHUNK, _DH), jnp.float32),
            pltpu.VMEM((_CHUNK, _DH), jnp.float32),
            pltpu.VMEM_SHARED((_N_PAD, _DH), jnp.float32),
        ],
    )
    return f(xf_flat, w_halves, src3, dst3)


def kernel(node_features, node_attr, edge_src, edge_dst, edge_attr,
           edge_scalars, W_sc, W_lin1, W_lin2, W_alpha, fc_w0, fc_w1):
    fc_w1_h = jnp.stack([fc_w1[:, :_DH], fc_w1[:, _DH:]], axis=0)
    W_lin1_h = jnp.stack([W_lin1[:, :_DH], W_lin1[:, _DH:]], axis=0)
    w_halves = _edge_mlp(edge_scalars, edge_attr, fc_w0, fc_w1_h)
    xf_flat = _node_lin(node_features, node_attr, W_lin1_h)

    src3 = edge_src.reshape(_NS, _NCHUNKS, _CHUNK)
    dst3 = edge_dst.reshape(_NS, _NCHUNKS, _CHUNK)

    parts = _sc_gather_scatter(xf_flat, w_halves, src3, dst3)
    return _final(node_features, node_attr, parts, W_sc, W_lin2, W_alpha)


# speedup vs baseline: 2.0709x; 1.3859x over previous
import jax
import jax.numpy as jnp
import numpy as np
from jax import lax
from jax.experimental import pallas as pl
from jax.experimental.pallas import tpu as pltpu
from jax.experimental.pallas import tpu_sc as plsc

_z = np.random.RandomState(0).randn(1_000_000)
_silu_np = _z / (1.0 + np.exp(-_z))
_SILU_CST = float(1.0 / np.sqrt(np.mean(_silu_np ** 2)))

_N = 10000
_E = 320000
_D = 128
_DH = 64
_DS = 16
_FH = 64
_INV_SQRT_DS = 1.0 / np.sqrt(float(_DS))
_INV_SQRT_FH = 1.0 / np.sqrt(float(_FH))
_INV_SQRT_D = 1.0 / np.sqrt(float(_D))
_INV_SQRT_NB = 1.0 / np.sqrt(32.0)

_NC = 2
_NS = 16
_E_PER_SUB = _E // _NS
_CHUNK = 80
_NCHUNKS = _E_PER_SUB // _CHUNK
_N_PAD = 10240
_ROWS_PER_SUB = _N_PAD // _NS


_BE = 2000
_BCH = _BE // _CHUNK
_GJ = _E_PER_SUB // _BE


def _edge_mlp_body(es_ref, ea_ref, w0_ref, w1_ref, out_ref):
    h = jnp.dot(es_ref[...], w0_ref[...] * _INV_SQRT_DS,
                preferred_element_type=jnp.float32)
    h = jax.nn.silu(h) * _SILU_CST
    ea = ea_ref[...] * _INV_SQRT_NB
    w0 = jnp.dot(h, w1_ref[0] * _INV_SQRT_FH,
                 preferred_element_type=jnp.float32) * ea
    w1 = jnp.dot(h, w1_ref[1] * _INV_SQRT_FH,
                 preferred_element_type=jnp.float32) * ea
    out_ref[0, 0] = w0.reshape(_BCH, _CHUNK, _DH)
    out_ref[1, 0] = w1.reshape(_BCH, _CHUNK, _DH)


def _edge_mlp(edge_scalars, edge_attr, fc_w0, fc_w1_halves):
    return pl.pallas_call(
        _edge_mlp_body,
        grid=(_NS, _GJ),
        in_specs=[
            pl.BlockSpec((_BE, _DS), lambda s, j: (s * _GJ + j, 0)),
            pl.BlockSpec((_BE, 1), lambda s, j: (s * _GJ + j, 0)),
            pl.BlockSpec((_DS, _FH), lambda s, j: (0, 0)),
            pl.BlockSpec((_NC, _FH, _DH), lambda s, j: (0, 0, 0)),
        ],
        out_specs=pl.BlockSpec((_NC, 1, _BCH, _CHUNK, _DH),
                               lambda s, j: (0, s, j, 0, 0)),
        out_shape=jax.ShapeDtypeStruct((_NC, _NS, _NCHUNKS, _CHUNK, _DH),
                                       jnp.float32),
    )(edge_scalars, edge_attr, fc_w0, fc_w1_halves)


def _node_lin_body(nf_ref, na_ref, w_ref, out_ref):
    xf = jnp.dot(nf_ref[...], w_ref[0], preferred_element_type=jnp.float32)
    out_ref[...] = xf * na_ref[...] * _INV_SQRT_D


def _node_lin(node_features, node_attr, W_lin1_halves):
    bn = 2000
    gi = _N // bn
    return pl.pallas_call(
        _node_lin_body,
        grid=(_NC, gi),
        in_specs=[
            pl.BlockSpec((bn, _D), lambda c, i: (i, 0)),
            pl.BlockSpec((bn, 1), lambda c, i: (i, 0)),
            pl.BlockSpec((1, _D, _DH), lambda c, i: (c, 0, 0)),
        ],
        out_specs=pl.BlockSpec((bn, _DH), lambda c, i: (c * gi + i, 0)),
        out_shape=jax.ShapeDtypeStruct((_NC * _N, _DH), jnp.float32),
    )(node_features, node_attr, W_lin1_halves)


def _final_body(nf_ref, na_ref, p_ref, wsc_ref, wl2_ref, wal_ref, out_ref):
    na = na_ref[...]
    p0 = p_ref[0]
    p1 = p_ref[1]
    nsc = jnp.dot(nf_ref[...], wsc_ref[...],
                  preferred_element_type=jnp.float32) * na * _INV_SQRT_D
    wl2 = wl2_ref[...]
    nco = (jnp.dot(p0, wl2[:_DH, :], preferred_element_type=jnp.float32)
           + jnp.dot(p1, wl2[_DH:, :], preferred_element_type=jnp.float32))
    nco = nco * na * _INV_SQRT_D
    wal = wal_ref[...]
    alpha = (jnp.dot(p0, wal[:_DH, :], preferred_element_type=jnp.float32)
             + jnp.dot(p1, wal[_DH:, :], preferred_element_type=jnp.float32))
    alpha = alpha * na * _INV_SQRT_D
    out_ref[...] = jax.nn.silu(nsc + alpha * nco) * _SILU_CST


def _final(node_features, node_attr, parts, W_sc, W_lin2, W_alpha):
    bn = 2000
    grid = _N // bn
    return pl.pallas_call(
        _final_body,
        grid=(grid,),
        in_specs=[
            pl.BlockSpec((bn, _D), lambda i: (i, 0)),
            pl.BlockSpec((bn, 1), lambda i: (i, 0)),
            pl.BlockSpec((_NC, bn, _DH), lambda i: (0, i, 0)),
            pl.BlockSpec((_D, _D), lambda i: (0, 0)),
            pl.BlockSpec((_D, _D), lambda i: (0, 0)),
            pl.BlockSpec((_D, 1), lambda i: (0, 0)),
        ],
        out_specs=pl.BlockSpec((bn, _D), lambda i: (i, 0)),
        out_shape=jax.ShapeDtypeStruct((_N, _D), jnp.float32),
    )(node_features, node_attr, parts, W_sc, W_lin2, W_alpha)


_NBUF = 2


def _sc_body(xf_hbm, w_hbm, src_hbm, dst_hbm, out_hbm,
             src_v, dst_v, rows_v, wchunk_v, zrow_v, agg_sh,
             sg0, sg1, sw0, sw1):
    cid = lax.axis_index("c")
    sid = lax.axis_index("s")
    sg = (sg0, sg1)
    sw = (sw0, sw1)

    pltpu.sync_copy(src_hbm.at[sid], src_v)
    pltpu.sync_copy(dst_hbm.at[sid], dst_v)

    off = cid * _N

    def off_body(c, carry):
        for j in range(_CHUNK // 16):
            sl = pl.ds(j * 16, 16)
            src_v[c, sl] = src_v[c, sl] + off
        return carry

    lax.fori_loop(0, _NCHUNKS, off_body, 0)

    for b in range(_NBUF):
        pltpu.async_copy(xf_hbm.at[src_v.at[b]], rows_v.at[b], sg[b])
        pltpu.async_copy(w_hbm.at[cid, sid, b], wchunk_v.at[b], sw[b])

    def z_body(e, carry):
        for j in range(_DH // 16):
            zrow_v[e, pl.ds(j * 16, 16)] = jnp.zeros((16,), jnp.float32)
        return carry

    lax.fori_loop(0, _CHUNK, z_body, 0)
    row0 = sid * _ROWS_PER_SUB
    for k in range(_ROWS_PER_SUB // _CHUNK):
        pltpu.sync_copy(zrow_v, agg_sh.at[pl.ds(row0 + k * _CHUNK, _CHUNK)])
    plsc.subcore_barrier()

    def do_chunk(c, b, prefetch):
        pltpu.make_async_copy(xf_hbm.at[src_v.at[c]], rows_v.at[b],
                              sg[b]).wait()
        pltpu.make_async_copy(w_hbm.at[cid, sid, c], wchunk_v.at[b],
                              sw[b]).wait()

        def mul_body(e, carry2):
            for j in range(_DH // 16):
                sl = pl.ds(j * 16, 16)
                rows_v[b, e, sl] = rows_v[b, e, sl] * wchunk_v[b, e, sl]
            return carry2

        lax.fori_loop(0, _CHUNK, mul_body, 0)

        pltpu.sync_copy(rows_v.at[b], agg_sh.at[dst_v.at[c]], add=True)

        if prefetch:
            nxt = c + _NBUF
            pltpu.async_copy(xf_hbm.at[src_v.at[nxt]], rows_v.at[b], sg[b])
            pltpu.async_copy(w_hbm.at[cid, sid, nxt], wchunk_v.at[b], sw[b])

    def chunk_body(g, carry):
        c0 = g * _NBUF
        for b in range(_NBUF):
            do_chunk(c0 + b, b, True)
        return carry

    lax.fori_loop(0, (_NCHUNKS - _NBUF) // _NBUF, chunk_body, 0)
    for b in range(_NBUF):
        do_chunk(_NCHUNKS - _NBUF + b, b, False)
    plsc.subcore_barrier()

    pltpu.sync_copy(agg_sh.at[pl.ds(row0, _ROWS_PER_SUB)],
                    out_hbm.at[cid, pl.ds(row0, _ROWS_PER_SUB)])


def _sc_gather_scatter(xf_flat, w_halves, src3, dst3):
    mesh = plsc.VectorSubcoreMesh(core_axis_name="c", subcore_axis_name="s")
    f = pl.kernel(
        _sc_body,
        out_type=jax.ShapeDtypeStruct((_NC, _N_PAD, _DH), jnp.float32),
        mesh=mesh,
        compiler_params=pltpu.CompilerParams(use_tc_tiling_on_sc=False),
        scratch_types=[
            pltpu.VMEM((_NCHUNKS, _CHUNK), jnp.int32),
            pltpu.VMEM((_NCHUNKS, _CHUNK), jnp.int32),
            pltpu.VMEM((_NBUF, _CHUNK, _DH), jnp.float32),
            pltpu.VMEM((_NBUF, _CHUNK, _DH), jnp.float32),
            pltpu.VMEM((_CHUNK, _DH), jnp.float32),
            pltpu.VMEM_SHARED((_N_PAD, _DH), jnp.float32),
            pltpu.SemaphoreType.DMA,
            pltpu.SemaphoreType.DMA,
            pltpu.SemaphoreType.DMA,
            pltpu.SemaphoreType.DMA,
        ],
    )
    return f(xf_flat, w_halves, src3, dst3)


def kernel(node_features, node_attr, edge_src, edge_dst, edge_attr,
           edge_scalars, W_sc, W_lin1, W_lin2, W_alpha, fc_w0, fc_w1):
    fc_w1_h = jnp.stack([fc_w1[:, :_DH], fc_w1[:, _DH:]], axis=0)
    W_lin1_h = jnp.stack([W_lin1[:, :_DH], W_lin1[:, _DH:]], axis=0)
    w_halves = _edge_mlp(edge_scalars, edge_attr, fc_w0, fc_w1_h)
    xf_flat = _node_lin(node_features, node_attr, W_lin1_h)

    src3 = edge_src.reshape(_NS, _NCHUNKS, _CHUNK)
    dst3 = edge_dst.reshape(_NS, _NCHUNKS, _CHUNK)

    parts = _sc_gather_scatter(xf_flat, w_halves, src3, dst3)
    return _final(node_features, node_attr, parts, W_sc, W_lin2, W_alpha)
